# Initial kernel scaffold; baseline (speedup 1.0000x reference)
import jax
import jax.numpy as jnp
from jax.experimental import pallas as pl


def kernel(x, species, abilities, items, movesets, teratypes):
    raise NotImplementedError("write your pallas kernel here")


# speedup vs baseline: 1.1893x; 1.1893x over previous
import functools

import jax
import jax.numpy as jnp
from jax import lax
from jax.experimental import pallas as pl
from jax.experimental.pallas import tpu as pltpu
from jax.experimental.pallas import tpu_sc as plsc

NC = 2
NS = 16
NW = NC * NS
B = 24576
BPW = B // NW


def _make_gather(Dp: int, R: int):
    nchunks = BPW // R
    mesh = plsc.VectorSubcoreMesh(core_axis_name="c", subcore_axis_name="s")

    @functools.partial(
        pl.kernel,
        out_type=jax.ShapeDtypeStruct((B, Dp), jnp.float32),
        mesh=mesh,
        scratch_types=[
            pltpu.VMEM((R,), jnp.int32),
            pltpu.VMEM((R, Dp), jnp.float32),
            pltpu.SemaphoreType.DMA,
        ],
        compiler_params=pltpu.CompilerParams(use_tc_tiling_on_sc=False),
    )
    def k(idx_hbm, table_hbm, out_hbm, idx_c, rows_v, sem):
        wid = lax.axis_index("s") * NC + lax.axis_index("c")
        base = wid * BPW

        def body(c, carry):
            off = base + pl.multiple_of(c * R, 8)
            pltpu.sync_copy(idx_hbm.at[pl.ds(off, R)], idx_c)
            pltpu.async_copy(table_hbm.at[idx_c], rows_v, sem).wait()
            pltpu.sync_copy(rows_v, out_hbm.at[pl.ds(off, R)])
            return carry

        lax.fori_loop(0, nchunks, body, 0)

    return k


def _pad8(d: int) -> int:
    return (d + 7) // 8 * 8


_CHUNK = {2047: 24, 1023: 48, 511: 96, 19: 128}
_KERNELS = {D: _make_gather(_pad8(D), R) for D, R in _CHUNK.items()}


def kernel(x, species, abilities, items, movesets, teratypes):
    idx = x.reshape(-1).astype(jnp.int32)
    outs = []
    for table in (species, abilities, items, movesets, teratypes):
        D = table.shape[1]
        Dp = _pad8(D)
        tp = table if Dp == D else jnp.pad(table, ((0, 0), (0, Dp - D)))
        out = _KERNELS[D](idx, tp)
        outs.append(out[:, :D].reshape(x.shape[0], x.shape[1], D))
    return tuple(outs)
